# Initial kernel scaffold; baseline (speedup 1.0000x reference)
import jax
import jax.numpy as jnp
from jax.experimental import pallas as pl


def kernel(x, W):
    raise NotImplementedError("write your pallas kernel here")


# speedup vs baseline: 3.3897x; 3.3897x over previous
import jax
import jax.numpy as jnp
from jax.experimental import pallas as pl

_BS = 512


def _emb_t_kernel(w_ref, o_ref):
    o_ref[...] = w_ref[...].T


def kernel(x, W):
    S = x.shape[0]
    N, D = W.shape
    return pl.pallas_call(
        _emb_t_kernel,
        grid=(S // _BS,),
        in_specs=[pl.BlockSpec((_BS, D), lambda j: (j, 0))],
        out_specs=pl.BlockSpec((D, _BS), lambda j: (0, j)),
        out_shape=jax.ShapeDtypeStruct((D, S), W.dtype),
    )(W)


# speedup vs baseline: 3.8326x; 1.1307x over previous
import jax
import jax.numpy as jnp
from jax.experimental import pallas as pl

_BS = 1024


def _emb_t_kernel(w_ref, o_ref):
    o_ref[...] = w_ref[...].T


def kernel(x, W):
    S = x.shape[0]
    N, D = W.shape
    return pl.pallas_call(
        _emb_t_kernel,
        grid=(S // _BS,),
        in_specs=[pl.BlockSpec((_BS, D), lambda j: (j, 0))],
        out_specs=pl.BlockSpec((D, _BS), lambda j: (0, j)),
        out_shape=jax.ShapeDtypeStruct((D, S), W.dtype),
    )(W)


# speedup vs baseline: 5.5307x; 1.4431x over previous
import jax
import jax.numpy as jnp
from jax.experimental import pallas as pl

_BS = 2048


def _emb_t_kernel(w_ref, o_ref):
    o_ref[...] = w_ref[...].T


def kernel(x, W):
    S = x.shape[0]
    N, D = W.shape
    return pl.pallas_call(
        _emb_t_kernel,
        grid=(S // _BS,),
        in_specs=[pl.BlockSpec((_BS, D), lambda j: (j, 0))],
        out_specs=pl.BlockSpec((D, _BS), lambda j: (0, j)),
        out_shape=jax.ShapeDtypeStruct((D, S), W.dtype),
    )(W)
